# baseline (speedup 1.0000x reference)
import numpy as np
import jax
import jax.numpy as jnp
from jax.experimental import pallas as pl
from jax.experimental.pallas import tpu as pltpu

C, T, V = 64, 2048, 128
LOW, HIGH = 512, 1024

_rng = np.random.default_rng(0)
_num = int(_rng.integers(LOW, HIGH + 1))
_masked_inds = np.asarray(_rng.choice(T, _num, replace=False), dtype=np.int64)
_fm = np.ones((T,), dtype=np.float32)
_fm[_masked_inds] = 0.0


_BR = 512


def _body(mask_ref, fm_ref, out_ref, fm_vmem):
    @pl.when(pl.program_id(0) == 0)
    def _init():
        fm_vmem[...] = jnp.broadcast_to(fm_ref[...], (T, V))

    i = pl.program_id(0)
    j = jax.lax.rem(i, T // _BR)
    out_ref[...] = mask_ref[...] * fm_vmem[pl.ds(j * _BR, _BR), :]


def kernel(x, mask):
    fmcol = jnp.asarray(_fm[:, None])
    m2d = mask.reshape(C * T, V)
    out = pl.pallas_call(
        _body,
        grid=(C * T // _BR,),
        in_specs=[
            pl.BlockSpec((_BR, V), lambda i: (i, 0)),
            pl.BlockSpec((T, 1), lambda i: (0, 0)),
        ],
        out_specs=pl.BlockSpec((_BR, V), lambda i: (i, 0)),
        out_shape=jax.ShapeDtypeStruct((C * T, V), jnp.float32),
        scratch_shapes=[pltpu.VMEM((T, V), jnp.float32)],
    )(m2d, fmcol)
    return (x, out.reshape(C, T, V))


# speedup vs baseline: 2.2203x; 2.2203x over previous
import numpy as np
import jax
import jax.numpy as jnp
from jax.experimental import pallas as pl
from jax.experimental.pallas import tpu as pltpu

C, T, V = 64, 2048, 128
LOW, HIGH = 512, 1024

_rng = np.random.default_rng(0)
_num = int(_rng.integers(LOW, HIGH + 1))
_masked_inds = np.asarray(_rng.choice(T, _num, replace=False), dtype=np.int64)
_fm = np.ones((T,), dtype=np.float32)
_fm[_masked_inds] = 0.0


_BR = 4096


def _body(mask_ref, fm_ref, out_ref, fm_vmem):
    @pl.when(pl.program_id(0) == 0)
    def _init():
        fm_vmem[...] = jnp.broadcast_to(fm_ref[...], (_BR, V))

    out_ref[...] = mask_ref[...] * fm_vmem[...]


def kernel(x, mask):
    fmcol = jnp.asarray(np.tile(_fm, _BR // T)[:, None])
    m2d = mask.reshape(C * T, V)
    out = pl.pallas_call(
        _body,
        grid=(C * T // _BR,),
        in_specs=[
            pl.BlockSpec((_BR, V), lambda i: (i, 0)),
            pl.BlockSpec((_BR, 1), lambda i: (0, 0)),
        ],
        out_specs=pl.BlockSpec((_BR, V), lambda i: (i, 0)),
        out_shape=jax.ShapeDtypeStruct((C * T, V), jnp.float32),
        scratch_shapes=[pltpu.VMEM((_BR, V), jnp.float32)],
    )(m2d, fmcol)
    return (x, out.reshape(C, T, V))


# speedup vs baseline: 2.2890x; 1.0309x over previous
import numpy as np
import jax
import jax.numpy as jnp
from jax.experimental import pallas as pl
from jax.experimental.pallas import tpu as pltpu

C, T, V = 64, 2048, 128
LOW, HIGH = 512, 1024

_rng = np.random.default_rng(0)
_num = int(_rng.integers(LOW, HIGH + 1))
_masked_inds = np.asarray(_rng.choice(T, _num, replace=False), dtype=np.int64)
_fm = np.ones((T,), dtype=np.float32)
_fm[_masked_inds] = 0.0


_BR = 8192


def _body(mask_ref, fm_ref, out_ref, fm_vmem):
    @pl.when(pl.program_id(0) == 0)
    def _init():
        fm_vmem[...] = jnp.broadcast_to(fm_ref[...], (_BR, V))

    out_ref[...] = mask_ref[...] * fm_vmem[...]


def kernel(x, mask):
    fmcol = jnp.asarray(np.tile(_fm, _BR // T)[:, None])
    m2d = mask.reshape(C * T, V)
    out = pl.pallas_call(
        _body,
        grid=(C * T // _BR,),
        in_specs=[
            pl.BlockSpec((_BR, V), lambda i: (i, 0)),
            pl.BlockSpec((_BR, 1), lambda i: (0, 0)),
        ],
        out_specs=pl.BlockSpec((_BR, V), lambda i: (i, 0)),
        out_shape=jax.ShapeDtypeStruct((C * T, V), jnp.float32),
        scratch_shapes=[pltpu.VMEM((_BR, V), jnp.float32)],
    )(m2d, fmcol)
    return (x, out.reshape(C, T, V))
